# Initial kernel scaffold; baseline (speedup 1.0000x reference)
import jax
import jax.numpy as jnp
from jax.experimental import pallas as pl


def kernel(input, weight):
    raise NotImplementedError("write your pallas kernel here")


# speedup vs baseline: 1.6961x; 1.6961x over previous
import functools

import jax
import jax.numpy as jnp
from jax import lax
from jax.experimental import pallas as pl
from jax.experimental.pallas import tpu as pltpu
from jax.experimental.pallas import tpu_sc as plsc

_NC = 2
_NS = 16
_NW = _NC * _NS

_B = 16384 * 50
_D = 64
_CHUNK = 128
_B_PER_W = _B // _NW
_NCHUNK = _B_PER_W // _CHUNK


def _embed_body(table_hbm, idx_hbm, out_hbm, idx_v, rows_v, gsem):
    wid = lax.axis_index("s") * _NC + lax.axis_index("c")
    pltpu.sync_copy(idx_hbm.at[wid], idx_v)

    def chunk(j, _):
        pltpu.async_copy(table_hbm.at[idx_v.at[j]], rows_v, gsem).wait()
        pltpu.sync_copy(rows_v, out_hbm.at[wid, j])
        return 0

    lax.fori_loop(0, _NCHUNK, chunk, 0)


@jax.jit
def _embed(idx, table):
    mesh = plsc.VectorSubcoreMesh(core_axis_name="c", subcore_axis_name="s")
    run = pl.kernel(
        _embed_body,
        out_type=jax.ShapeDtypeStruct((_NW, _NCHUNK, _CHUNK, _D), jnp.float32),
        mesh=mesh,
        compiler_params=pltpu.CompilerParams(use_tc_tiling_on_sc=False),
        scratch_types=[
            pltpu.VMEM((_NCHUNK, _CHUNK), jnp.int32),
            pltpu.VMEM((_CHUNK, _D), jnp.float32),
            pltpu.SemaphoreType.DMA,
        ],
    )
    return run(table, idx)


def kernel(input, weight):
    idx = input.reshape(_NW, _NCHUNK, _CHUNK).astype(jnp.int32)
    out = _embed(idx, weight)
    return out.reshape(input.shape + (_D,))


# speedup vs baseline: 1.8750x; 1.1055x over previous
import jax
import jax.numpy as jnp
from jax import lax
from jax.experimental import pallas as pl
from jax.experimental.pallas import tpu as pltpu
from jax.experimental.pallas import tpu_sc as plsc

_NC = 2
_NS = 16
_NW = _NC * _NS

_B = 16384 * 50
_D = 64
_CHUNK = 128
_B_PER_W = _B // _NW
_NCHUNK = _B_PER_W // _CHUNK
_K = 5
_NROUND = _NCHUNK // _K


def _embed_body(table_hbm, idx_hbm, out_hbm, idx_v, rows_v, gsem, ssem):
    wid = lax.axis_index("s") * _NC + lax.axis_index("c")
    pltpu.sync_copy(idx_hbm.at[wid], idx_v)

    def fire_gathers(r, grp):
        for b in range(_K):
            pltpu.make_async_copy(
                table_hbm.at[idx_v.at[r * _K + b]], rows_v.at[grp, b], gsem
            ).start()

    def wait_gathers(grp):
        for b in range(_K):
            pltpu.make_async_copy(
                table_hbm.at[idx_v.at[0]], rows_v.at[grp, b], gsem
            ).wait()

    def fire_stores(r, grp):
        for b in range(_K):
            pltpu.make_async_copy(
                rows_v.at[grp, b], out_hbm.at[wid, r * _K + b], ssem
            ).start()

    def wait_stores(r, grp):
        for b in range(_K):
            pltpu.make_async_copy(
                rows_v.at[grp, b], out_hbm.at[wid, r * _K + b], ssem
            ).wait()

    fire_gathers(0, 0)

    def round_body(r, _):
        grp = lax.rem(r, 2)
        wait_gathers(grp)
        fire_gathers(r + 1, 1 - grp)
        fire_stores(r, grp)
        wait_stores(r, grp)
        return 0

    lax.fori_loop(0, _NROUND - 1, round_body, 0)

    grp = (_NROUND - 1) % 2
    wait_gathers(grp)
    fire_stores(_NROUND - 1, grp)
    wait_stores(_NROUND - 1, grp)


@jax.jit
def _embed(idx, table):
    mesh = plsc.VectorSubcoreMesh(core_axis_name="c", subcore_axis_name="s")
    run = pl.kernel(
        _embed_body,
        out_type=jax.ShapeDtypeStruct((_NW, _NCHUNK, _CHUNK, _D), jnp.float32),
        mesh=mesh,
        compiler_params=pltpu.CompilerParams(use_tc_tiling_on_sc=False),
        scratch_types=[
            pltpu.VMEM((_NCHUNK, _CHUNK), jnp.int32),
            pltpu.VMEM((2, _K, _CHUNK, _D), jnp.float32),
            pltpu.SemaphoreType.DMA,
            pltpu.SemaphoreType.DMA,
        ],
    )
    return run(table, idx)


def kernel(input, weight):
    idx = input.reshape(_NW, _NCHUNK, _CHUNK).astype(jnp.int32)
    out = _embed(idx, weight)
    return out.reshape(input.shape + (_D,))


# speedup vs baseline: 2.5170x; 1.3424x over previous
import jax
import jax.numpy as jnp
from jax import lax
from jax.experimental import pallas as pl
from jax.experimental.pallas import tpu as pltpu
from jax.experimental.pallas import tpu_sc as plsc

_NC = 2
_NS = 16
_NW = _NC * _NS

_V = 1_000_000
_B = 16384 * 50
_D = 64
_CHUNK = 128
_B_PER_W = _B // _NW
_NCHUNK = _B_PER_W // _CHUNK
_K = 5
_NROUND = _NCHUNK // _K

_TW = 8192
_TG = 123
_VP = _TG * _TW
_MAIN = (_TG - 1) * _TW


def _transpose_body(wt_ref, out_ref):
    y = wt_ref[...].T
    out_ref[...] = jnp.concatenate([y[: _TW // 2], y[_TW // 2 :]], axis=1)


@jax.jit
def _relayout(wt):
    return pl.pallas_call(
        _transpose_body,
        grid=(_TG,),
        in_specs=[pl.BlockSpec((_D, _TW), lambda c: (0, c))],
        out_specs=pl.BlockSpec((_TW // 2, 2 * _D), lambda c: (c, 0)),
        out_shape=jax.ShapeDtypeStruct((_VP // 2, 2 * _D), jnp.float32),
    )(wt)


def _embed_body(table_hbm, idx_hbm, out_hbm, idx_v, rows_v, gsem, ssem):
    wid = lax.axis_index("s") * _NC + lax.axis_index("c")
    pltpu.sync_copy(idx_hbm.at[wid], idx_v)

    def fire_gathers(r, grp):
        for b in range(_K):
            pltpu.make_async_copy(
                table_hbm.at[idx_v.at[r * _K + b]], rows_v.at[grp, b], gsem
            ).start()

    def wait_gathers(grp):
        for b in range(_K):
            pltpu.make_async_copy(
                table_hbm.at[idx_v.at[0]], rows_v.at[grp, b], gsem
            ).wait()

    def fire_stores(r, grp):
        for b in range(_K):
            pltpu.make_async_copy(
                rows_v.at[grp, b], out_hbm.at[wid, r * _K + b], ssem
            ).start()

    def wait_stores(r, grp):
        for b in range(_K):
            pltpu.make_async_copy(
                rows_v.at[grp, b], out_hbm.at[wid, r * _K + b], ssem
            ).wait()

    fire_gathers(0, 0)

    def round_body(r, _):
        grp = lax.rem(r, 2)
        wait_gathers(grp)
        fire_gathers(r + 1, 1 - grp)
        fire_stores(r, grp)
        wait_stores(r, grp)
        return 0

    lax.fori_loop(0, _NROUND - 1, round_body, 0)

    grp = (_NROUND - 1) % 2
    wait_gathers(grp)
    fire_stores(_NROUND - 1, grp)
    wait_stores(_NROUND - 1, grp)


@jax.jit
def _embed(idx, table):
    mesh = plsc.VectorSubcoreMesh(core_axis_name="c", subcore_axis_name="s")
    run = pl.kernel(
        _embed_body,
        out_type=jax.ShapeDtypeStruct((_NW, _NCHUNK, _CHUNK, _D), jnp.float32),
        mesh=mesh,
        compiler_params=pltpu.CompilerParams(use_tc_tiling_on_sc=False),
        scratch_types=[
            pltpu.VMEM((_NCHUNK, _CHUNK), jnp.int32),
            pltpu.VMEM((2, _K, _CHUNK, _D), jnp.float32),
            pltpu.SemaphoreType.DMA,
            pltpu.SemaphoreType.DMA,
        ],
    )
    return run(table, idx)


def kernel(input, weight):
    i = input.reshape(_NW, _NCHUNK, _CHUNK).astype(jnp.int32)
    half = _TW // 2
    u_main = (i // _TW) * _TW + 2 * (i % half) + (i // half) % 2
    idx = jnp.where(i < _MAIN, u_main, 2 * i - _MAIN)
    table = _relayout(weight.T).reshape(_VP, _D)
    out = _embed(idx, table)
    return out.reshape(input.shape + (_D,))


# speedup vs baseline: 2.9844x; 1.1857x over previous
import jax
import jax.numpy as jnp
from jax import lax
from jax.experimental import pallas as pl
from jax.experimental.pallas import tpu as pltpu
from jax.experimental.pallas import tpu_sc as plsc

_NC = 2
_NS = 16
_NW = _NC * _NS

_V = 1_000_000
_B = 16384 * 50
_D = 64
_CHUNK = 128
_B_PER_W = _B // _NW
_NCHUNK = _B_PER_W // _CHUNK
_K = 5
_NROUND = _NCHUNK // _K

_TW = 8192
_TG = 123
_VP = _TG * _TW
_MAIN = (_TG - 1) * _TW


def _transpose_body(wt_ref, out_ref):
    y = wt_ref[...].T
    out_ref[...] = jnp.concatenate([y[: _TW // 2], y[_TW // 2 :]], axis=1)


@jax.jit
def _relayout(wt):
    return pl.pallas_call(
        _transpose_body,
        grid=(_TG,),
        in_specs=[pl.BlockSpec((_D, _TW), lambda c: (0, c))],
        out_specs=pl.BlockSpec((_TW // 2, 2 * _D), lambda c: (c, 0)),
        out_shape=jax.ShapeDtypeStruct((_VP // 2, 2 * _D), jnp.float32),
    )(wt)


_A = 16384
_S = 50


def _untranspose_body(o_ref, l_ref):
    x = o_ref[0]
    t = x.T
    l_ref[0] = jnp.concatenate([t[: _D], t[_D :]], axis=1)


@jax.jit
def _untranspose(o3):
    return pl.pallas_call(
        _untranspose_body,
        grid=(_S,),
        in_specs=[pl.BlockSpec((1, _A // 2, 2 * _D), lambda s: (s, 0, 0))],
        out_specs=pl.BlockSpec((1, _D, _A), lambda s: (s, 0, 0)),
        out_shape=jax.ShapeDtypeStruct((_S, _D, _A), jnp.float32),
    )(o3)


def _embed_body(table_hbm, idx_hbm, out_hbm, idx_v, rows_v, gsem, ssem):
    wid = lax.axis_index("s") * _NC + lax.axis_index("c")
    pltpu.sync_copy(idx_hbm.at[wid], idx_v)

    def fire_gathers(r, grp):
        for b in range(_K):
            pltpu.make_async_copy(
                table_hbm.at[idx_v.at[r * _K + b]], rows_v.at[grp, b], gsem
            ).start()

    def wait_gathers(grp):
        for b in range(_K):
            pltpu.make_async_copy(
                table_hbm.at[idx_v.at[0]], rows_v.at[grp, b], gsem
            ).wait()

    def fire_stores(r, grp):
        for b in range(_K):
            pltpu.make_async_copy(
                rows_v.at[grp, b], out_hbm.at[wid, r * _K + b], ssem
            ).start()

    def wait_stores(r, grp):
        for b in range(_K):
            pltpu.make_async_copy(
                rows_v.at[grp, b], out_hbm.at[wid, r * _K + b], ssem
            ).wait()

    fire_gathers(0, 0)

    def round_body(r, _):
        grp = lax.rem(r, 2)
        wait_gathers(grp)
        fire_gathers(r + 1, 1 - grp)
        fire_stores(r, grp)
        wait_stores(r, grp)
        return 0

    lax.fori_loop(0, _NROUND - 1, round_body, 0)

    grp = (_NROUND - 1) % 2
    wait_gathers(grp)
    fire_stores(_NROUND - 1, grp)
    wait_stores(_NROUND - 1, grp)


@jax.jit
def _embed(idx, table):
    mesh = plsc.VectorSubcoreMesh(core_axis_name="c", subcore_axis_name="s")
    run = pl.kernel(
        _embed_body,
        out_type=jax.ShapeDtypeStruct((_NW, _NCHUNK, _CHUNK, _D), jnp.float32),
        mesh=mesh,
        compiler_params=pltpu.CompilerParams(use_tc_tiling_on_sc=False),
        scratch_types=[
            pltpu.VMEM((_NCHUNK, _CHUNK), jnp.int32),
            pltpu.VMEM((2, _K, _CHUNK, _D), jnp.float32),
            pltpu.SemaphoreType.DMA,
            pltpu.SemaphoreType.DMA,
        ],
    )
    return run(table, idx)


def kernel(input, weight):
    i3 = input.T.reshape(_S, 2, _A // 2).transpose(0, 2, 1)
    i = i3.reshape(_NW, _NCHUNK, _CHUNK).astype(jnp.int32)
    half = _TW // 2
    u_main = (i // _TW) * _TW + 2 * (i % half) + (i // half) % 2
    idx = jnp.where(i < _MAIN, u_main, 2 * i - _MAIN)
    table = _relayout(weight.T).reshape(_VP, _D)
    out = _embed(idx, table)
    l = _untranspose(out.reshape(_S, _A // 2, 2 * _D))
    return l.transpose(2, 0, 1)


# speedup vs baseline: 3.1141x; 1.0435x over previous
import jax
import jax.numpy as jnp
from jax import lax
from jax.experimental import pallas as pl
from jax.experimental.pallas import tpu as pltpu
from jax.experimental.pallas import tpu_sc as plsc

_NC = 2
_NS = 16
_NW = _NC * _NS

_V = 1_000_000
_B = 16384 * 50
_D = 64
_CHUNK = 128
_B_PER_W = _B // _NW
_NCHUNK = _B_PER_W // _CHUNK
_K = 5
_NROUND = _NCHUNK // _K

_TW = 16384
_TG = 62
_VP = _TG * _TW
_MAIN = (_TG - 1) * _TW


def _transpose_body(wt_ref, out_ref):
    y = wt_ref[...].T
    out_ref[...] = jnp.concatenate([y[: _TW // 2], y[_TW // 2 :]], axis=1)


@jax.jit
def _relayout(wt):
    return pl.pallas_call(
        _transpose_body,
        grid=(_TG,),
        in_specs=[pl.BlockSpec((_D, _TW), lambda c: (0, c))],
        out_specs=pl.BlockSpec((_TW // 2, 2 * _D), lambda c: (c, 0)),
        out_shape=jax.ShapeDtypeStruct((_VP // 2, 2 * _D), jnp.float32),
    )(wt)


_A = 16384
_S = 50


def _untranspose_body(o_ref, l_ref):
    x = o_ref[0]
    t = x.T
    l_ref[0] = jnp.concatenate([t[: _D], t[_D :]], axis=1)


@jax.jit
def _untranspose(o3):
    return pl.pallas_call(
        _untranspose_body,
        grid=(_S,),
        in_specs=[pl.BlockSpec((1, _A // 2, 2 * _D), lambda s: (s, 0, 0))],
        out_specs=pl.BlockSpec((1, _D, _A), lambda s: (s, 0, 0)),
        out_shape=jax.ShapeDtypeStruct((_S, _D, _A), jnp.float32),
    )(o3)


def _embed_body(table_hbm, idx_hbm, out_hbm, idx_v, rows_v, gsem, ssem):
    wid = lax.axis_index("s") * _NC + lax.axis_index("c")
    pltpu.sync_copy(idx_hbm.at[wid], idx_v)

    def fire_gathers(r, grp):
        for b in range(_K):
            pltpu.make_async_copy(
                table_hbm.at[idx_v.at[r * _K + b]], rows_v.at[grp, b], gsem
            ).start()

    def wait_gathers(grp):
        for b in range(_K):
            pltpu.make_async_copy(
                table_hbm.at[idx_v.at[0]], rows_v.at[grp, b], gsem
            ).wait()

    def fire_stores(r, grp):
        for b in range(_K):
            pltpu.make_async_copy(
                rows_v.at[grp, b], out_hbm.at[wid, r * _K + b], ssem
            ).start()

    def wait_stores(r, grp):
        for b in range(_K):
            pltpu.make_async_copy(
                rows_v.at[grp, b], out_hbm.at[wid, r * _K + b], ssem
            ).wait()

    fire_gathers(0, 0)

    def round_body(r, _):
        grp = lax.rem(r, 2)
        wait_gathers(grp)
        fire_gathers(r + 1, 1 - grp)
        fire_stores(r, grp)
        wait_stores(r, grp)
        return 0

    lax.fori_loop(0, _NROUND - 1, round_body, 0)

    grp = (_NROUND - 1) % 2
    wait_gathers(grp)
    fire_stores(_NROUND - 1, grp)
    wait_stores(_NROUND - 1, grp)


@jax.jit
def _embed(idx, table):
    mesh = plsc.VectorSubcoreMesh(core_axis_name="c", subcore_axis_name="s")
    run = pl.kernel(
        _embed_body,
        out_type=jax.ShapeDtypeStruct((_NW, _NCHUNK, _CHUNK, _D), jnp.float32),
        mesh=mesh,
        compiler_params=pltpu.CompilerParams(use_tc_tiling_on_sc=False),
        scratch_types=[
            pltpu.VMEM((_NCHUNK, _CHUNK), jnp.int32),
            pltpu.VMEM((2, _K, _CHUNK, _D), jnp.float32),
            pltpu.SemaphoreType.DMA,
            pltpu.SemaphoreType.DMA,
        ],
    )
    return run(table, idx)


def kernel(input, weight):
    i3 = input.T.reshape(_S, 2, _A // 2).transpose(0, 2, 1)
    i = i3.reshape(_NW, _NCHUNK, _CHUNK).astype(jnp.int32)
    half = _TW // 2
    u_main = (i // _TW) * _TW + 2 * (i % half) + (i // half) % 2
    idx = jnp.where(i < _MAIN, u_main, 2 * i - _MAIN)
    table = _relayout(weight.T).reshape(_VP, _D)
    out = _embed(idx, table)
    l = _untranspose(out.reshape(_S, _A // 2, 2 * _D))
    return l.transpose(2, 0, 1)


# speedup vs baseline: 4.4793x; 1.4384x over previous
import jax
import jax.numpy as jnp
from jax import lax
from jax.experimental import pallas as pl
from jax.experimental.pallas import tpu as pltpu
from jax.experimental.pallas import tpu_sc as plsc

_NC = 2
_NS = 16
_NW = _NC * _NS

_V = 1_000_000
_B = 16384 * 50
_D = 64
_CHUNK = 128
_B_PER_W = _B // _NW
_NCHUNK = _B_PER_W // _CHUNK
_K = 5
_NROUND = _NCHUNK // _K

_TW = 16384
_TG = 62
_VP = _TG * _TW
_MAIN = (_TG - 1) * _TW


def _transpose_body(wt_ref, out_ref):
    y = wt_ref[...].T
    out_ref[...] = jnp.concatenate([y[: _TW // 2], y[_TW // 2 :]], axis=1)


@jax.jit
def _relayout(wt):
    return pl.pallas_call(
        _transpose_body,
        grid=(_TG,),
        in_specs=[pl.BlockSpec((_D, _TW), lambda c: (0, c))],
        out_specs=pl.BlockSpec((_TW // 2, 2 * _D), lambda c: (c, 0)),
        out_shape=jax.ShapeDtypeStruct((_VP // 2, 2 * _D), jnp.float32),
    )(wt)


_A = 16384
_S = 50


def _untranspose_body(o_ref, l_ref):
    x = o_ref[0]
    t = x.T
    l_ref[0] = jnp.concatenate([t[: _D], t[_D :]], axis=1)


@jax.jit
def _untranspose(o3):
    return pl.pallas_call(
        _untranspose_body,
        grid=(_S,),
        in_specs=[pl.BlockSpec((1, _A // 2, 2 * _D), lambda s: (s, 0, 0))],
        out_specs=pl.BlockSpec((1, _D, _A), lambda s: (s, 0, 0)),
        out_shape=jax.ShapeDtypeStruct((_S, _D, _A), jnp.float32),
    )(o3)


def _embed_body(table_hbm, idx_hbm, out_hbm, idx_v, rows_v, gsem, ssem):
    wid = lax.axis_index("s") * _NC + lax.axis_index("c")

    def stage(l, _):
        gc = wid * _NCHUNK + l
        sp = gc // (_A // _CHUNK)
        c = lax.rem(gc, _A // _CHUNK)
        pltpu.make_async_copy(
            idx_hbm.at[sp, 0, c], idx_v.at[l, pl.ds(0, _D)], gsem
        ).start()
        pltpu.make_async_copy(
            idx_hbm.at[sp, 1, c], idx_v.at[l, pl.ds(_D, _D)], gsem
        ).start()
        return 0

    def stage_wait(l, _):
        gc = wid * _NCHUNK + l
        sp = gc // (_A // _CHUNK)
        c = lax.rem(gc, _A // _CHUNK)
        pltpu.make_async_copy(
            idx_hbm.at[sp, 0, c], idx_v.at[l, pl.ds(0, _D)], gsem
        ).wait()
        pltpu.make_async_copy(
            idx_hbm.at[sp, 1, c], idx_v.at[l, pl.ds(_D, _D)], gsem
        ).wait()
        return 0

    lax.fori_loop(0, _NCHUNK, stage, 0)
    lax.fori_loop(0, _NCHUNK, stage_wait, 0)

    def fire_gathers(r, grp):
        for b in range(_K):
            pltpu.make_async_copy(
                table_hbm.at[idx_v.at[r * _K + b]], rows_v.at[grp, b], gsem
            ).start()

    def wait_gathers(grp):
        for b in range(_K):
            pltpu.make_async_copy(
                table_hbm.at[idx_v.at[0]], rows_v.at[grp, b], gsem
            ).wait()

    def fire_stores(r, grp):
        for b in range(_K):
            for h in range(2):
                pltpu.make_async_copy(
                    rows_v.at[grp, b, pl.ds(h * _D, _D)],
                    out_hbm.at[wid, r * _K + b, :, h],
                    ssem,
                ).start()

    def wait_stores(r, grp):
        for b in range(_K):
            for h in range(2):
                pltpu.make_async_copy(
                    rows_v.at[grp, b, pl.ds(h * _D, _D)],
                    out_hbm.at[wid, r * _K + b, :, h],
                    ssem,
                ).wait()

    fire_gathers(0, 0)

    def round_body(r, _):
        grp = lax.rem(r, 2)
        wait_gathers(grp)
        fire_gathers(r + 1, 1 - grp)
        fire_stores(r, grp)
        wait_stores(r, grp)
        return 0

    lax.fori_loop(0, _NROUND - 1, round_body, 0)

    grp = (_NROUND - 1) % 2
    wait_gathers(grp)
    fire_stores(_NROUND - 1, grp)
    wait_stores(_NROUND - 1, grp)


@jax.jit
def _embed(idx, table):
    mesh = plsc.VectorSubcoreMesh(core_axis_name="c", subcore_axis_name="s")
    run = pl.kernel(
        _embed_body,
        out_type=jax.ShapeDtypeStruct((_NW, _NCHUNK, _D, 2, _D), jnp.float32),
        mesh=mesh,
        compiler_params=pltpu.CompilerParams(use_tc_tiling_on_sc=False),
        scratch_types=[
            pltpu.VMEM((_NCHUNK, _CHUNK), jnp.int32),
            pltpu.VMEM((2, _K, _CHUNK, _D), jnp.float32),
            pltpu.SemaphoreType.DMA,
            pltpu.SemaphoreType.DMA,
        ],
    )
    return run(table, idx)


def kernel(input, weight):
    it = input.T.astype(jnp.int32)
    half = _TW // 2
    u_main = (it // _TW) * _TW + 2 * (it % half) + (it // half) % 2
    iv = jnp.where(it < _MAIN, u_main, 2 * it - _MAIN)
    idx = iv.reshape(_S, 2, _A // _CHUNK, _D)
    table = _relayout(weight.T).reshape(_VP, _D)
    out = _embed(idx, table)
    l = _untranspose(out.reshape(_S, _A // 2, 2 * _D))
    return l.transpose(2, 0, 1)
